# baseline (speedup 1.0000x reference)
import functools

import jax
import jax.numpy as jnp
from jax import lax
from jax.experimental import pallas as pl
from jax.experimental.pallas import tpu as pltpu
from jax.experimental.pallas import tpu_sc as plsc

L = 16
CH = 128
CT = 7813
PLANE = CT * 1024
HI_MASK = -65536


def _stage_base(idx, base, nch, ngr):
    def b_body(t, carry):
        j = t // ngr
        jj = t % ngr
        sl = pl.ds(pl.multiple_of(jj * L, L), L)
        v = idx[j, sl]
        base[j, sl] = ((v >> 7) << 10) | (v & 127)
        return carry

    lax.fori_loop(0, nch * ngr, b_body, 0)


def _issue_gathers(tab, base, pidx, rT, sem, dp, nch, ngr):
    def d_body(kk, carry):
        dbase = (kk >> 3) * PLANE + (kk & 7) * CH

        def p_body(t, c2):
            j = t // ngr
            jj = t % ngr
            sl = pl.ds(pl.multiple_of(jj * L, L), L)
            pidx[kk, j, sl] = base[j, sl] + dbase
            return c2

        lax.fori_loop(0, nch * ngr, p_body, 0)
        for j in range(nch):
            pltpu.async_copy(tab.at[pidx.at[kk, j]], rT.at[kk, j], sem)
        return carry

    lax.fori_loop(0, dp, d_body, 0)


def _gather_body(nc, bpw, dp, ut_f, uidx_h, uw_h,
                 uidx, ubase, pidxu, urT, sem):
    wid = lax.axis_index("s") * nc + lax.axis_index("c")
    pltpu.sync_copy(uidx_h.at[wid], uidx)
    nch = bpw // CH
    ngr = CH // L
    _stage_base(uidx, ubase, nch, ngr)
    _issue_gathers(ut_f, ubase, pidxu, urT, sem, dp, nch, ngr)
    pltpu.make_async_copy(uidx_h.at[pl.ds(0, dp)], urT, sem).wait()
    pltpu.sync_copy(urT, uw_h.at[wid])


def _combine_body(nc, bpw, dp, mt_f, midx_h, uw_h, wb_h, out_h,
                  midx, mbase, pidxm, urT, mrT, wbv, outv, sem):
    wid = lax.axis_index("s") * nc + lax.axis_index("c")
    pltpu.sync_copy(wb_h, wbv)
    pltpu.sync_copy(midx_h.at[wid], midx)
    pltpu.sync_copy(uw_h.at[wid], urT)
    nch = bpw // CH
    ngr = CH // L
    _stage_base(midx, mbase, nch, ngr)
    _issue_gathers(mt_f, mbase, pidxm, mrT, sem, dp, nch, ngr)
    pltpu.make_async_copy(midx_h.at[pl.ds(0, dp)], mrT, sem).wait()

    wvecs = [wbv[pl.ds(i * L, L)] for i in range((2 * dp) // L)]
    ws = [wvecs[k // L][k % L] for k in range(2 * dp)]
    bias = wbv[pl.ds(pl.multiple_of(2 * dp, L), L)][0]

    def g_body(g, carry):
        j = g >> 3
        sl = pl.ds(pl.multiple_of((g & 7) * L, L), L)
        acc = jnp.zeros((L,), jnp.float32)
        for k in range(dp):
            vu = urT[k, j, sl]
            vm = mrT[k, j, sl]
            u0 = plsc.bitcast(vu << 16, jnp.float32)
            u1 = plsc.bitcast(vu & HI_MASK, jnp.float32)
            m0 = plsc.bitcast(vm << 16, jnp.float32)
            m1 = plsc.bitcast(vm & HI_MASK, jnp.float32)
            acc = acc + u0 * m0 * ws[2 * k] + u1 * m1 * ws[2 * k + 1]
        x = acc + bias
        y = 1.0 / (1.0 + jnp.exp(-x))
        outv[pl.ds(pl.multiple_of(g * L, L), L)] = y
        return carry

    lax.fori_loop(0, bpw // L, g_body, 0)
    pltpu.sync_copy(outv, out_h.at[wid])


PACK_B = 1664


def _pack_body(in_ref, out_ref):
    u = lax.bitcast_convert_type(in_ref[...], jnp.int32)
    r = u + 0x7FFF + ((u >> 16) & 1)
    d2 = r.shape[0] // 2
    out_ref[...] = (r[d2:, :] & HI_MASK) | ((r[:d2, :] >> 16) & 0xFFFF)


def _flat_packed(table):
    n, d = table.shape
    npad = CT * 128
    out = pl.pallas_call(
        _pack_body,
        grid=(npad // PACK_B,),
        in_specs=[pl.BlockSpec((d, PACK_B), lambda i: (0, i))],
        out_specs=pl.BlockSpec((d // 2, PACK_B), lambda i: (0, i)),
        out_shape=jax.ShapeDtypeStruct((d // 2, npad), jnp.int32),
    )(table.T)
    return (out.reshape((d // 2) // 8, 8, CT, 128)
               .transpose(0, 2, 1, 3)
               .reshape(-1))


def kernel(users, movies, user_table, movie_table, W, b):
    batch = users.shape[0]
    d = user_table.shape[1]
    dp = d // 2

    info = plsc.get_sparse_core_info()
    nc, ns = info.num_cores, info.num_subcores
    nw = nc * ns
    bpw = batch // nw

    users3 = users.astype(jnp.int32).reshape(nw, bpw // CH, CH)
    movies3 = movies.astype(jnp.int32).reshape(nw, bpw // CH, CH)
    perm = []
    for kk in range(dp):
        perm += [kk, kk + dp]
    wb = jnp.concatenate([W.reshape(-1)[jnp.array(perm)], b.reshape(-1),
                          jnp.zeros((15,), jnp.float32)])

    mesh = plsc.VectorSubcoreMesh(core_axis_name="c", subcore_axis_name="s")
    cparams = pltpu.CompilerParams(needs_layout_passes=False,
                                   use_tc_tiling_on_sc=False)

    ut_f = _flat_packed(user_table)
    run_gather = pl.kernel(
        functools.partial(_gather_body, nc, bpw, dp),
        out_type=jax.ShapeDtypeStruct((nw, dp, bpw // CH, CH), jnp.int32),
        mesh=mesh,
        compiler_params=cparams,
        scratch_types=[
            pltpu.VMEM((bpw // CH, CH), jnp.int32),
            pltpu.VMEM((bpw // CH, CH), jnp.int32),
            pltpu.VMEM((dp, bpw // CH, CH), jnp.int32),
            pltpu.VMEM((dp, bpw // CH, CH), jnp.int32),
            pltpu.SemaphoreType.DMA,
        ],
    )
    uw = run_gather(ut_f, users3)

    mt_f = _flat_packed(movie_table)
    run_combine = pl.kernel(
        functools.partial(_combine_body, nc, bpw, dp),
        out_type=jax.ShapeDtypeStruct((nw, bpw), jnp.float32),
        mesh=mesh,
        compiler_params=cparams,
        scratch_types=[
            pltpu.VMEM((bpw // CH, CH), jnp.int32),
            pltpu.VMEM((bpw // CH, CH), jnp.int32),
            pltpu.VMEM((dp, bpw // CH, CH), jnp.int32),
            pltpu.VMEM((dp, bpw // CH, CH), jnp.int32),
            pltpu.VMEM((dp, bpw // CH, CH), jnp.int32),
            pltpu.VMEM((d + 16,), jnp.float32),
            pltpu.VMEM((bpw,), jnp.float32),
            pltpu.SemaphoreType.DMA,
        ],
    )
    out = run_combine(mt_f, movies3, uw, wb)
    return out.reshape(batch, 1)


# speedup vs baseline: 4.4704x; 4.4704x over previous
import functools

import jax
import jax.numpy as jnp
from jax import lax
from jax.experimental import pallas as pl
from jax.experimental.pallas import tpu as pltpu
from jax.experimental.pallas import tpu_sc as plsc

L = 16
CH = 128
CT = 7813
PLANE = CT * 1024
HI_MASK = -65536


def _stage_base(idx, base, nch, ngr):
    def b_body(t, carry):
        j = t // ngr
        jj = t % ngr
        sl = pl.ds(pl.multiple_of(jj * L, L), L)
        v = idx[j, sl]
        base[j, sl] = ((v >> 7) << 10) | (v & 127)
        return carry

    lax.fori_loop(0, nch * ngr, b_body, 0)


def _issue_gathers(tab, base, pidx, rT, sem, dp, nch, ngr):
    def d_body(kk, carry):
        dbase = (kk >> 3) * PLANE + (kk & 7) * CH

        def p_body(t, c2):
            j = t // ngr
            jj = t % ngr
            sl = pl.ds(pl.multiple_of(jj * L, L), L)
            pidx[kk, j, sl] = base[j, sl] + dbase
            return c2

        lax.fori_loop(0, nch * ngr, p_body, 0)
        for j in range(nch):
            pltpu.async_copy(tab.at[pidx.at[kk, j]], rT.at[kk, j], sem)
        return carry

    lax.fori_loop(0, dp, d_body, 0)


def _gather_body(nc, bpw, dp, ut_f, uidx_h, uw_h,
                 uidx, ubase, pidxu, urT, sem):
    wid = lax.axis_index("s") * nc + lax.axis_index("c")
    pltpu.sync_copy(uidx_h.at[wid], uidx)
    nch = bpw // CH
    ngr = CH // L
    _stage_base(uidx, ubase, nch, ngr)
    _issue_gathers(ut_f, ubase, pidxu, urT, sem, dp, nch, ngr)
    pltpu.make_async_copy(uidx_h.at[pl.ds(0, dp)], urT, sem).wait()
    pltpu.sync_copy(urT, uw_h.at[wid])


def _combine_body(nc, bpw, dp, mt_f, midx_h, uw_h, wb_h, out_h,
                  midx, mbase, pidxm, urT, mrT, wbv, outv, sem):
    wid = lax.axis_index("s") * nc + lax.axis_index("c")
    pltpu.sync_copy(wb_h, wbv)
    pltpu.sync_copy(midx_h.at[wid], midx)
    pltpu.sync_copy(uw_h.at[wid], urT)
    nch = bpw // CH
    ngr = CH // L
    _stage_base(midx, mbase, nch, ngr)
    _issue_gathers(mt_f, mbase, pidxm, mrT, sem, dp, nch, ngr)
    pltpu.make_async_copy(midx_h.at[pl.ds(0, dp)], mrT, sem).wait()

    wvecs = [wbv[pl.ds(i * L, L)] for i in range((2 * dp) // L)]
    ws = [wvecs[k // L][k % L] for k in range(2 * dp)]
    bias = wbv[pl.ds(pl.multiple_of(2 * dp, L), L)][0]

    def g_body(g, carry):
        j = g >> 3
        sl = pl.ds(pl.multiple_of((g & 7) * L, L), L)
        acc = jnp.zeros((L,), jnp.float32)
        for k in range(dp):
            vu = urT[k, j, sl]
            vm = mrT[k, j, sl]
            u0 = plsc.bitcast(vu << 16, jnp.float32)
            u1 = plsc.bitcast(vu & HI_MASK, jnp.float32)
            m0 = plsc.bitcast(vm << 16, jnp.float32)
            m1 = plsc.bitcast(vm & HI_MASK, jnp.float32)
            acc = acc + u0 * m0 * ws[2 * k] + u1 * m1 * ws[2 * k + 1]
        x = acc + bias
        y = 1.0 / (1.0 + jnp.exp(-x))
        outv[pl.ds(pl.multiple_of(g * L, L), L)] = y
        return carry

    lax.fori_loop(0, bpw // L, g_body, 0)
    pltpu.sync_copy(outv, out_h.at[wid])


PACK_B = 76928


def _pack_body(in_ref, out_ref):
    u = lax.bitcast_convert_type(in_ref[...], jnp.int32)
    r = u + 0x7FFF + ((u >> 16) & 1)
    d2 = r.shape[0] // 2
    out_ref[...] = (r[d2:, :] & HI_MASK) | ((r[:d2, :] >> 16) & 0xFFFF)


def _flat_packed(table):
    n, d = table.shape
    npad = CT * 128
    out = pl.pallas_call(
        _pack_body,
        grid=(npad // PACK_B,),
        in_specs=[pl.BlockSpec((d, PACK_B), lambda i: (0, i))],
        out_specs=pl.BlockSpec((d // 2, PACK_B), lambda i: (0, i)),
        out_shape=jax.ShapeDtypeStruct((d // 2, npad), jnp.int32),
    )(table.T)
    return (out.reshape((d // 2) // 8, 8, CT, 128)
               .transpose(0, 2, 1, 3)
               .reshape(-1))


def kernel(users, movies, user_table, movie_table, W, b):
    batch = users.shape[0]
    d = user_table.shape[1]
    dp = d // 2

    info = plsc.get_sparse_core_info()
    nc, ns = info.num_cores, info.num_subcores
    nw = nc * ns
    bpw = batch // nw

    users3 = users.astype(jnp.int32).reshape(nw, bpw // CH, CH)
    movies3 = movies.astype(jnp.int32).reshape(nw, bpw // CH, CH)
    perm = []
    for kk in range(dp):
        perm += [kk, kk + dp]
    wb = jnp.concatenate([W.reshape(-1)[jnp.array(perm)], b.reshape(-1),
                          jnp.zeros((15,), jnp.float32)])

    mesh = plsc.VectorSubcoreMesh(core_axis_name="c", subcore_axis_name="s")
    cparams = pltpu.CompilerParams(needs_layout_passes=False,
                                   use_tc_tiling_on_sc=False)

    ut_f = _flat_packed(user_table)
    run_gather = pl.kernel(
        functools.partial(_gather_body, nc, bpw, dp),
        out_type=jax.ShapeDtypeStruct((nw, dp, bpw // CH, CH), jnp.int32),
        mesh=mesh,
        compiler_params=cparams,
        scratch_types=[
            pltpu.VMEM((bpw // CH, CH), jnp.int32),
            pltpu.VMEM((bpw // CH, CH), jnp.int32),
            pltpu.VMEM((dp, bpw // CH, CH), jnp.int32),
            pltpu.VMEM((dp, bpw // CH, CH), jnp.int32),
            pltpu.SemaphoreType.DMA,
        ],
    )
    uw = run_gather(ut_f, users3)

    mt_f = _flat_packed(movie_table)
    run_combine = pl.kernel(
        functools.partial(_combine_body, nc, bpw, dp),
        out_type=jax.ShapeDtypeStruct((nw, bpw), jnp.float32),
        mesh=mesh,
        compiler_params=cparams,
        scratch_types=[
            pltpu.VMEM((bpw // CH, CH), jnp.int32),
            pltpu.VMEM((bpw // CH, CH), jnp.int32),
            pltpu.VMEM((dp, bpw // CH, CH), jnp.int32),
            pltpu.VMEM((dp, bpw // CH, CH), jnp.int32),
            pltpu.VMEM((dp, bpw // CH, CH), jnp.int32),
            pltpu.VMEM((d + 16,), jnp.float32),
            pltpu.VMEM((bpw,), jnp.float32),
            pltpu.SemaphoreType.DMA,
        ],
    )
    out = run_combine(mt_f, movies3, uw, wb)
    return out.reshape(batch, 1)
